# baseline (speedup 1.0000x reference)
import functools

import jax
import jax.numpy as jnp
from jax import lax
from jax.experimental import pallas as pl
from jax.experimental.pallas import tpu as pltpu
from jax.experimental.pallas import tpu_sc as plsc

S = 2048
HID = 1024
INTER = 2048
E = 8
K = 2
N = S * K
BLK = 256
G = N // BLK + E
NSLOTS = G * BLK

NWORKERS = 32
ROWS_PER_WORKER = NSLOTS // NWORKERS
GATHER_CHUNK = 32
NCHUNK = ROWS_PER_WORKER // GATHER_CHUNK
NBUF = 3
TOK_PER_WORKER = S // NWORKERS
TOK_CHUNK = 16


def _routing_metadata(expert_indices, expert_weights):
    idx_flat = expert_indices.reshape(N).astype(jnp.int32)
    w_flat = expert_weights.reshape(N)
    oh = (idx_flat[:, None] == jnp.arange(E, dtype=jnp.int32)[None, :])
    ohi = oh.astype(jnp.int32)
    incl = jnp.cumsum(ohi, axis=0)
    rank = jnp.sum((incl - ohi) * ohi, axis=1)
    counts = incl[-1]
    nblk = (counts + BLK - 1) // BLK
    blk_starts = jnp.concatenate([jnp.zeros((1,), jnp.int32),
                                  jnp.cumsum(nblk)[:-1].astype(jnp.int32)])
    pad_start = blk_starts * BLK
    pair_slot = pad_start[idx_flat] + rank
    block_expert = jnp.clip(
        jnp.searchsorted(blk_starts, jnp.arange(G, dtype=jnp.int32), side="right")
        .astype(jnp.int32) - 1, 0, E - 1)
    r = jnp.arange(N, dtype=jnp.int32)
    pad_needed = nblk * BLK - counts
    pad_cum = jnp.concatenate([jnp.zeros((1,), jnp.int32),
                               jnp.cumsum(pad_needed)[:-1].astype(jnp.int32)])
    t = jnp.arange(NSLOTS - N, dtype=jnp.int32)
    pe = jnp.clip(jnp.sum(t[:, None] >= pad_cum[None, :], axis=1)
                  .astype(jnp.int32) - 1, 0, E - 1)
    keys = jnp.concatenate([idx_flat * (2 * N) + r,
                            pe * (2 * N) + N + t])
    toks = jnp.concatenate([r // K, t % S])
    ws = jnp.concatenate([w_flat, jnp.zeros((NSLOTS - N,), jnp.float32)])
    _, gather_tok, slot_w = lax.sort((keys, toks, ws), num_keys=1)
    return gather_tok, slot_w, block_expert, pair_slot


def _sc_gather_body(x_hbm, ids_hbm, out_hbm, idx_v, bufs, gsems, wsems):
    wid = lax.axis_index("s") * 2 + lax.axis_index("c")
    base = wid * ROWS_PER_WORKER
    pltpu.sync_copy(ids_hbm.at[pl.ds(base, ROWS_PER_WORKER)], idx_v)

    def start_gather(c):
        b = c % NBUF
        return pltpu.async_copy(
            x_hbm.at[idx_v.at[pl.ds(c * GATHER_CHUNK, GATHER_CHUNK)]],
            bufs.at[b], gsems.at[b])

    def start_write(c):
        b = c % NBUF
        return pltpu.async_copy(
            bufs.at[b], out_hbm.at[pl.ds(base + c * GATHER_CHUNK, GATHER_CHUNK)],
            wsems.at[b])

    g = [None] * NCHUNK
    w = [None] * NCHUNK
    g[0] = start_gather(0)
    for c in range(NCHUNK):
        if c + 1 < NCHUNK:
            if c + 1 >= NBUF:
                w[c + 1 - NBUF].wait()
            g[c + 1] = start_gather(c + 1)
        g[c].wait()
        w[c] = start_write(c)
    for c in range(max(0, NCHUNK - NBUF), NCHUNK):
        w[c].wait()


def _sc_combine_body(y_hbm, pair_hbm, out_hbm, idx_v, rows_v, acc_v, gsems, wsems):
    wid = lax.axis_index("s") * 2 + lax.axis_index("c")
    tbase = wid * TOK_PER_WORKER
    nc = TOK_PER_WORKER // TOK_CHUNK
    pltpu.sync_copy(pair_hbm.at[pl.ds(tbase * K, TOK_PER_WORKER * K)], idx_v)

    def start_gather(c):
        b = c % 2
        return pltpu.async_copy(
            y_hbm.at[idx_v.at[pl.ds(c * K * TOK_CHUNK, K * TOK_CHUNK)]],
            rows_v.at[b], gsems.at[b])

    def start_write(c):
        b = c % 2
        return pltpu.async_copy(
            acc_v.at[b], out_hbm.at[pl.ds(tbase + c * TOK_CHUNK, TOK_CHUNK)],
            wsems.at[b])

    g = [None] * nc
    w = [None] * nc
    g[0] = start_gather(0)
    for c in range(nc):
        if c + 1 < nc:
            g[c + 1] = start_gather(c + 1)
        g[c].wait()
        if c >= 2:
            w[c - 2].wait()
        b = c % 2

        def col_body(j, carry):
            cs = pl.ds(j * 16, 16)
            for i in range(TOK_CHUNK):
                acc_v[b, i, cs] = rows_v[b, 2 * i, cs] + rows_v[b, 2 * i + 1, cs]
            return carry

        lax.fori_loop(0, HID // 16, col_body, 0)
        w[c] = start_write(c)
    for c in range(max(0, nc - 2), nc):
        w[c].wait()


@functools.lru_cache(maxsize=None)
def _build_sc_kernels():
    mesh = plsc.VectorSubcoreMesh(core_axis_name="c", subcore_axis_name="s")
    gather = pl.kernel(
        _sc_gather_body,
        out_type=jax.ShapeDtypeStruct((NSLOTS, HID // 2), jnp.int32),
        mesh=mesh,
        scratch_types=[
            pltpu.VMEM((ROWS_PER_WORKER,), jnp.int32),
            pltpu.VMEM((NBUF, GATHER_CHUNK, HID // 2), jnp.int32),
            pltpu.SemaphoreType.DMA((NBUF,)),
            pltpu.SemaphoreType.DMA((NBUF,)),
        ],
    )
    combine = pl.kernel(
        _sc_combine_body,
        out_type=jax.ShapeDtypeStruct((S, HID), jnp.float32),
        mesh=mesh,
        scratch_types=[
            pltpu.VMEM((K * TOK_PER_WORKER,), jnp.int32),
            pltpu.VMEM((2, K * TOK_CHUNK, HID), jnp.float32),
            pltpu.VMEM((2, TOK_CHUNK, HID), jnp.float32),
            pltpu.SemaphoreType.DMA((2,)),
            pltpu.SemaphoreType.DMA((2,)),
        ],
    )
    return gather, combine


def _ffn_body(be_ref, xs_ref, g_ref, u_ref, d_ref, w_ref, o_ref):
    xb = xs_ref[...]
    gate = lax.dot_general(xb, g_ref[0].astype(jnp.bfloat16),
                           (((1,), (1,)), ((), ())),
                           preferred_element_type=jnp.float32)
    up = lax.dot_general(xb, u_ref[0].astype(jnp.bfloat16),
                         (((1,), (1,)), ((), ())),
                         preferred_element_type=jnp.float32)
    h = ((gate * jax.nn.sigmoid(gate)) * up).astype(jnp.bfloat16)
    y = lax.dot_general(h, d_ref[0].astype(jnp.bfloat16),
                        (((1,), (1,)), ((), ())),
                        preferred_element_type=jnp.float32)
    o_ref[...] = y * w_ref[0, 0][:, None]


def _tc_ffn(x_sorted, gate_proj, up_proj, down_proj, slot_w, block_expert):
    grid_spec = pltpu.PrefetchScalarGridSpec(
        num_scalar_prefetch=1,
        grid=(G,),
        in_specs=[
            pl.BlockSpec((BLK, HID), lambda g, be: (g, 0)),
            pl.BlockSpec((1, INTER, HID), lambda g, be: (be[g], 0, 0)),
            pl.BlockSpec((1, INTER, HID), lambda g, be: (be[g], 0, 0)),
            pl.BlockSpec((1, HID, INTER), lambda g, be: (be[g], 0, 0)),
            pl.BlockSpec((1, 1, BLK), lambda g, be: (g, 0, 0)),
        ],
        out_specs=pl.BlockSpec((BLK, HID), lambda g, be: (g, 0)),
    )
    return pl.pallas_call(
        _ffn_body,
        grid_spec=grid_spec,
        out_shape=jax.ShapeDtypeStruct((NSLOTS, HID), jnp.float32),
        compiler_params=pltpu.CompilerParams(
            vmem_limit_bytes=63 * 1024 * 1024,
        ),
    )(block_expert, x_sorted, gate_proj, up_proj, down_proj,
      slot_w.reshape(G, 1, BLK))


def kernel(x, expert_indices, expert_weights, gate_proj, up_proj, down_proj):
    batch, seq, hid = x.shape
    x_bf = x.reshape(S, HID).astype(jnp.bfloat16)
    x_i32 = lax.bitcast_convert_type(x_bf.reshape(S, HID // 2, 2), jnp.int32)
    gather_tok, slot_w, block_expert, pair_slot = _routing_metadata(
        expert_indices, expert_weights)
    sc_gather, sc_combine = _build_sc_kernels()
    x_sorted_i32 = sc_gather(x_i32, gather_tok)
    x_sorted = lax.bitcast_convert_type(
        x_sorted_i32, jnp.bfloat16).reshape(NSLOTS, HID)
    y_sorted = _tc_ffn(x_sorted, gate_proj, up_proj, down_proj,
                       slot_w, block_expert)
    out = sc_combine(y_sorted, pair_slot)
    return out.reshape(batch, seq, hid)


# speedup vs baseline: 1.8378x; 1.8378x over previous
import functools

import jax
import jax.numpy as jnp
from jax import lax
from jax.experimental import pallas as pl
from jax.experimental.pallas import tpu as pltpu
from jax.experimental.pallas import tpu_sc as plsc

S = 2048
HID = 1024
INTER = 2048
E = 8
K = 2
N = S * K
BLK = 256
G = N // BLK + E
NSLOTS = G * BLK

NWORKERS = 32
ROWS_PER_WORKER = NSLOTS // NWORKERS
GATHER_CHUNK = 32
NCHUNK = ROWS_PER_WORKER // GATHER_CHUNK
NBUF = 3
TOK_PER_WORKER = S // NWORKERS
TOK_CHUNK = 16


def _routing_metadata(expert_indices, expert_weights):
    idx_flat = expert_indices.reshape(N).astype(jnp.int32)
    w_flat = expert_weights.reshape(N)
    oh = (idx_flat[:, None] == jnp.arange(E, dtype=jnp.int32)[None, :])
    ohi = oh.astype(jnp.int32)
    incl = jnp.cumsum(ohi, axis=0)
    rank = jnp.sum((incl - ohi) * ohi, axis=1)
    counts = incl[-1]
    nblk = (counts + BLK - 1) // BLK
    blk_starts = jnp.concatenate([jnp.zeros((1,), jnp.int32),
                                  jnp.cumsum(nblk)[:-1].astype(jnp.int32)])
    pad_start = blk_starts * BLK
    pair_slot = pad_start[idx_flat] + rank
    block_expert = jnp.clip(
        jnp.searchsorted(blk_starts, jnp.arange(G, dtype=jnp.int32), side="right")
        .astype(jnp.int32) - 1, 0, E - 1)
    r = jnp.arange(N, dtype=jnp.int32)
    pad_needed = nblk * BLK - counts
    pad_cum = jnp.concatenate([jnp.zeros((1,), jnp.int32),
                               jnp.cumsum(pad_needed)[:-1].astype(jnp.int32)])
    t = jnp.arange(NSLOTS - N, dtype=jnp.int32)
    pe = jnp.clip(jnp.sum(t[:, None] >= pad_cum[None, :], axis=1)
                  .astype(jnp.int32) - 1, 0, E - 1)
    keys = jnp.concatenate([idx_flat * (2 * N) + r,
                            pe * (2 * N) + N + t])
    toks = jnp.concatenate([r // K, t % S])
    ws = jnp.concatenate([w_flat, jnp.zeros((NSLOTS - N,), jnp.float32)])
    _, gather_tok, slot_w = lax.sort((keys, toks, ws), num_keys=1)
    return gather_tok, slot_w, block_expert, pair_slot


def _sc_gather_body(x_hbm, ids_hbm, out_hbm, idx_v, bufs, gsems, wsems):
    wid = lax.axis_index("s") * 2 + lax.axis_index("c")
    base = wid * ROWS_PER_WORKER
    pltpu.sync_copy(ids_hbm.at[pl.ds(base, ROWS_PER_WORKER)], idx_v)

    def start_gather(c):
        b = c % NBUF
        return pltpu.async_copy(
            x_hbm.at[idx_v.at[pl.ds(c * GATHER_CHUNK, GATHER_CHUNK)]],
            bufs.at[b], gsems.at[b])

    def start_write(c):
        b = c % NBUF
        return pltpu.async_copy(
            bufs.at[b], out_hbm.at[pl.ds(base + c * GATHER_CHUNK, GATHER_CHUNK)],
            wsems.at[b])

    g = [None] * NCHUNK
    w = [None] * NCHUNK
    g[0] = start_gather(0)
    for c in range(NCHUNK):
        if c + 1 < NCHUNK:
            if c + 1 >= NBUF:
                w[c + 1 - NBUF].wait()
            g[c + 1] = start_gather(c + 1)
        g[c].wait()
        w[c] = start_write(c)
    for c in range(max(0, NCHUNK - NBUF), NCHUNK):
        w[c].wait()


def _sc_combine_body(y_hbm, pair_hbm, out_hbm, idx_v, rows_v, acc_v, gsems, wsems):
    wid = lax.axis_index("s") * 2 + lax.axis_index("c")
    tbase = wid * TOK_PER_WORKER
    nc = TOK_PER_WORKER // TOK_CHUNK
    pltpu.sync_copy(pair_hbm.at[pl.ds(tbase * K, TOK_PER_WORKER * K)], idx_v)

    def start_gather(c):
        b = c % 2
        return pltpu.async_copy(
            y_hbm.at[idx_v.at[pl.ds(c * K * TOK_CHUNK, K * TOK_CHUNK)]],
            rows_v.at[b], gsems.at[b])

    def start_write(c):
        b = c % 2
        return pltpu.async_copy(
            acc_v.at[b], out_hbm.at[pl.ds(tbase + c * TOK_CHUNK, TOK_CHUNK)],
            wsems.at[b])

    g = [None] * nc
    w = [None] * nc
    g[0] = start_gather(0)
    for c in range(nc):
        if c + 1 < nc:
            g[c + 1] = start_gather(c + 1)
        g[c].wait()
        if c >= 2:
            w[c - 2].wait()
        b = c % 2

        def col_body(j, carry):
            cs = pl.ds(j * 16, 16)
            for i in range(TOK_CHUNK):
                acc_v[b, i, cs] = rows_v[b, 2 * i, cs] + rows_v[b, 2 * i + 1, cs]
            return carry

        lax.fori_loop(0, HID // 16, col_body, 0)
        w[c] = start_write(c)
    for c in range(max(0, nc - 2), nc):
        w[c].wait()


@functools.lru_cache(maxsize=None)
def _build_sc_kernels():
    mesh = plsc.VectorSubcoreMesh(core_axis_name="c", subcore_axis_name="s")
    gather = pl.kernel(
        _sc_gather_body,
        out_type=jax.ShapeDtypeStruct((NSLOTS, HID), jnp.float32),
        mesh=mesh,
        scratch_types=[
            pltpu.VMEM((ROWS_PER_WORKER,), jnp.int32),
            pltpu.VMEM((NBUF, GATHER_CHUNK, HID), jnp.float32),
            pltpu.SemaphoreType.DMA((NBUF,)),
            pltpu.SemaphoreType.DMA((NBUF,)),
        ],
    )
    combine = pl.kernel(
        _sc_combine_body,
        out_type=jax.ShapeDtypeStruct((S, HID), jnp.float32),
        mesh=mesh,
        scratch_types=[
            pltpu.VMEM((K * TOK_PER_WORKER,), jnp.int32),
            pltpu.VMEM((2, K * TOK_CHUNK, HID), jnp.float32),
            pltpu.VMEM((2, TOK_CHUNK, HID), jnp.float32),
            pltpu.SemaphoreType.DMA((2,)),
            pltpu.SemaphoreType.DMA((2,)),
        ],
    )
    return gather, combine


def _ffn_body(be_ref, xs_ref, g_ref, u_ref, d_ref, w_ref, o_ref):
    xb = xs_ref[...].astype(jnp.bfloat16)
    gate = lax.dot_general(xb, g_ref[0].astype(jnp.bfloat16),
                           (((1,), (1,)), ((), ())),
                           preferred_element_type=jnp.float32)
    up = lax.dot_general(xb, u_ref[0].astype(jnp.bfloat16),
                         (((1,), (1,)), ((), ())),
                         preferred_element_type=jnp.float32)
    h = ((gate * jax.nn.sigmoid(gate)) * up).astype(jnp.bfloat16)
    y = lax.dot_general(h, d_ref[0].astype(jnp.bfloat16),
                        (((1,), (1,)), ((), ())),
                        preferred_element_type=jnp.float32)
    o_ref[...] = y * w_ref[0, 0][:, None]


def _tc_ffn(x_sorted, gate_proj, up_proj, down_proj, slot_w, block_expert):
    grid_spec = pltpu.PrefetchScalarGridSpec(
        num_scalar_prefetch=1,
        grid=(G,),
        in_specs=[
            pl.BlockSpec((BLK, HID), lambda g, be: (g, 0)),
            pl.BlockSpec((1, INTER, HID), lambda g, be: (be[g], 0, 0)),
            pl.BlockSpec((1, INTER, HID), lambda g, be: (be[g], 0, 0)),
            pl.BlockSpec((1, HID, INTER), lambda g, be: (be[g], 0, 0)),
            pl.BlockSpec((1, 1, BLK), lambda g, be: (g, 0, 0)),
        ],
        out_specs=pl.BlockSpec((BLK, HID), lambda g, be: (g, 0)),
    )
    return pl.pallas_call(
        _ffn_body,
        grid_spec=grid_spec,
        out_shape=jax.ShapeDtypeStruct((NSLOTS, HID), jnp.float32),
        compiler_params=pltpu.CompilerParams(
            vmem_limit_bytes=63 * 1024 * 1024,
        ),
    )(block_expert, x_sorted, gate_proj, up_proj, down_proj,
      slot_w.reshape(G, 1, BLK))


def kernel(x, expert_indices, expert_weights, gate_proj, up_proj, down_proj):
    batch, seq, hid = x.shape
    x2d = x.reshape(S, HID)
    gather_tok, slot_w, block_expert, pair_slot = _routing_metadata(
        expert_indices, expert_weights)
    sc_gather, sc_combine = _build_sc_kernels()
    x_sorted = sc_gather(x2d, gather_tok)
    y_sorted = _tc_ffn(x_sorted, gate_proj, up_proj, down_proj,
                       slot_w, block_expert)
    out = sc_combine(y_sorted, pair_slot)
    return out.reshape(batch, seq, hid)
